# baseline (speedup 1.0000x reference)
import jax
import jax.numpy as jnp
from jax.experimental import pallas as pl
from jax.experimental.pallas import tpu as pltpu
from functools import partial

N, F, H, C = 50000, 512, 64, 16
BLOCK_N = 10000


def _split3_dot(a, b):
    ah = a.astype(jnp.bfloat16)
    al = (a - ah.astype(jnp.float32)).astype(jnp.bfloat16)
    bh = b.astype(jnp.bfloat16)
    bl = (b - bh.astype(jnp.float32)).astype(jnp.bfloat16)
    return (jnp.dot(ah, bh, preferred_element_type=jnp.float32)
            + jnp.dot(ah, bl, preferred_element_type=jnp.float32)
            + jnp.dot(al, bh, preferred_element_type=jnp.float32))


def _fused_kernel(x_ref, mask_ref, w_ref, w1t_ref, b1_ref, w2t_ref, b2_ref,
                  w3t_ref, b3_ref, out_ref):
    wm = w_ref[...] * mask_ref[...]
    wc = _split3_dot(wm, w1t_ref[...])
    h = jnp.dot(x_ref[...].astype(jnp.bfloat16), wc.astype(jnp.bfloat16),
                preferred_element_type=jnp.float32)
    h = jnp.maximum(h + b1_ref[...], 0.0)
    h = jnp.dot(h.astype(jnp.bfloat16), w2t_ref[...].astype(jnp.bfloat16),
                preferred_element_type=jnp.float32)
    h = jnp.maximum(h + b2_ref[...], 0.0)
    out_ref[...] = (
        jnp.dot(h.astype(jnp.bfloat16), w3t_ref[...].astype(jnp.bfloat16),
                preferred_element_type=jnp.float32)
        + b3_ref[...])


@jax.jit
def kernel(x, feature_mask, w, W1, b1, W2, b2, W3, b3):
    mask_f = feature_mask.astype(jnp.float32).reshape(F, 1)
    grid = (N + BLOCK_N - 1) // BLOCK_N
    full = lambda *s: pl.BlockSpec(s, lambda i: (0,) * len(s))
    return pl.pallas_call(
        _fused_kernel,
        grid=(grid,),
        in_specs=[
            pl.BlockSpec((BLOCK_N, F), lambda i: (i, 0)),
            full(F, 1),
            full(F, H),
            full(H, H),
            full(1, H),
            full(H, H),
            full(1, H),
            full(H, C),
            full(1, C),
        ],
        out_specs=pl.BlockSpec((BLOCK_N, C), lambda i: (i, 0)),
        out_shape=jax.ShapeDtypeStruct((N, C), jnp.float32),
    )(x, mask_f, w, W1.T, b1.reshape(1, H), W2.T, b2.reshape(1, H),
      W3.T, b3.reshape(1, C))


# speedup vs baseline: 1.0727x; 1.0727x over previous
import jax
import jax.numpy as jnp
from jax.experimental import pallas as pl
from jax.experimental.pallas import tpu as pltpu
from functools import partial

N, F, H, C = 50000, 512, 64, 16
BLOCK_N = 10000


def _split3_dot(a, b):
    ah = a.astype(jnp.bfloat16)
    al = (a - ah.astype(jnp.float32)).astype(jnp.bfloat16)
    bh = b.astype(jnp.bfloat16)
    bl = (b - bh.astype(jnp.float32)).astype(jnp.bfloat16)
    return (jnp.dot(ah, bh, preferred_element_type=jnp.float32)
            + jnp.dot(ah, bl, preferred_element_type=jnp.float32)
            + jnp.dot(al, bh, preferred_element_type=jnp.float32))


def _fused_kernel(x_ref, mask_ref, w_ref, w1t_ref, b1_ref, w2t_ref, b2_ref,
                  w3t_ref, b3_ref, out_ref, wc_ref):
    @pl.when(pl.program_id(0) == 0)
    def _():
        wm = w_ref[...] * mask_ref[...]
        wc_ref[...] = _split3_dot(wm, w1t_ref[...])

    h = jnp.dot(x_ref[...], wc_ref[...], preferred_element_type=jnp.float32)
    h = jnp.maximum(h + b1_ref[...], 0.0)
    h = jnp.maximum(
        jnp.dot(h, w2t_ref[...], preferred_element_type=jnp.float32)
        + b2_ref[...], 0.0)
    out_ref[...] = (
        jnp.dot(h, w3t_ref[...], preferred_element_type=jnp.float32)
        + b3_ref[...])


@jax.jit
def kernel(x, feature_mask, w, W1, b1, W2, b2, W3, b3):
    mask_f = feature_mask.astype(jnp.float32).reshape(F, 1)
    grid = (N + BLOCK_N - 1) // BLOCK_N
    full = lambda *s: pl.BlockSpec(s, lambda i: (0,) * len(s))
    return pl.pallas_call(
        _fused_kernel,
        grid=(grid,),
        in_specs=[
            pl.BlockSpec((BLOCK_N, F), lambda i: (i, 0)),
            full(F, 1),
            full(F, H),
            full(H, H),
            full(1, H),
            full(H, H),
            full(1, H),
            full(H, C),
            full(1, C),
        ],
        out_specs=pl.BlockSpec((BLOCK_N, C), lambda i: (i, 0)),
        out_shape=jax.ShapeDtypeStruct((N, C), jnp.float32),
        scratch_shapes=[pltpu.VMEM((F, H), jnp.float32)],
    )(x, mask_f, w, W1.T, b1.reshape(1, H), W2.T, b2.reshape(1, H),
      W3.T, b3.reshape(1, C))
